# baseline (speedup 1.0000x reference)
import jax
import jax.numpy as jnp
from jax.experimental import pallas as pl

_SLICES_PER_STEP = 4
_ROW_SPLIT = 2


def _gcn_body(x_ref, a_ref, w_ref, b_ref, o_ref):
    n = a_ref.shape[0]
    h = n // _ROW_SPLIT
    j = pl.program_id(1)
    a_rows = a_ref[pl.ds(j * h, h), :]
    for s in range(_SLICES_PER_STEP):
        tmp = jnp.dot(a_rows, x_ref[s].astype(jnp.bfloat16),
                      preferred_element_type=jnp.float32)
        out = jnp.dot(tmp.astype(jnp.bfloat16), w_ref[...],
                      preferred_element_type=jnp.float32)
        o_ref[s] = out + b_ref[...]


def kernel(inputs, normalized_adj, weights_0, bias_0):
    b, t, n, d = inputs.shape
    hid = weights_0.shape[1]
    s = _SLICES_PER_STEP
    h = n // _ROW_SPLIT
    x = inputs.reshape(b * t, n, d)
    a_bf = normalized_adj.astype(jnp.bfloat16)
    w_bf = weights_0.astype(jnp.bfloat16)
    bias2 = bias_0.reshape(1, hid)

    out = pl.pallas_call(
        _gcn_body,
        grid=(b * t // s, _ROW_SPLIT),
        in_specs=[
            pl.BlockSpec((s, n, d), lambda i, j: (i, 0, 0)),
            pl.BlockSpec((n, n), lambda i, j: (0, 0)),
            pl.BlockSpec((d, hid), lambda i, j: (0, 0)),
            pl.BlockSpec((1, hid), lambda i, j: (0, 0)),
        ],
        out_specs=pl.BlockSpec((s, h, hid), lambda i, j: (i, j, 0)),
        out_shape=jax.ShapeDtypeStruct((b * t, n, hid), jnp.float32),
    )(x, a_bf, w_bf, bias2)
    return out.reshape(b, t, n, hid)


# speedup vs baseline: 1.0757x; 1.0757x over previous
import jax
import jax.numpy as jnp
from jax.experimental import pallas as pl

_GROUP = 8
_SUB = 4


def _gcn_body(x_ref, a_ref, w_ref, b_ref, o_ref):
    j = pl.program_id(1)
    for s in range(_SUB):
        xb = x_ref[j * _SUB + s].astype(jnp.bfloat16)
        tmp = jnp.dot(a_ref[...], xb, preferred_element_type=jnp.float32)
        out = jnp.dot(tmp.astype(jnp.bfloat16), w_ref[...],
                      preferred_element_type=jnp.float32)
        o_ref[s] = out + b_ref[...]


def kernel(inputs, normalized_adj, weights_0, bias_0):
    b, t, n, d = inputs.shape
    hid = weights_0.shape[1]
    x = inputs.reshape(b * t, n, d)
    a_bf = normalized_adj.astype(jnp.bfloat16)
    w_bf = weights_0.astype(jnp.bfloat16)
    bias2 = bias_0.reshape(1, hid)
    nsub = _GROUP // _SUB

    out = pl.pallas_call(
        _gcn_body,
        grid=(b * t // _GROUP, nsub),
        in_specs=[
            pl.BlockSpec((_GROUP, n, d), lambda i, j: (i, 0, 0)),
            pl.BlockSpec((n, n), lambda i, j: (0, 0)),
            pl.BlockSpec((d, hid), lambda i, j: (0, 0)),
            pl.BlockSpec((1, hid), lambda i, j: (0, 0)),
        ],
        out_specs=pl.BlockSpec((_SUB, n, hid), lambda i, j: (i * nsub + j, 0, 0)),
        out_shape=jax.ShapeDtypeStruct((b * t, n, hid), jnp.float32),
    )(x, a_bf, w_bf, bias2)
    return out.reshape(b, t, n, hid)


# speedup vs baseline: 1.2829x; 1.1926x over previous
import jax
import jax.numpy as jnp
from jax.experimental import pallas as pl

_SLICES_PER_STEP = 4


def _gcn_body(x_ref, a_ref, w_ref, b_ref, o_ref):
    tmps = [
        jnp.dot(a_ref[...], x_ref[s].astype(jnp.bfloat16),
                preferred_element_type=jnp.float32).astype(jnp.bfloat16)
        for s in range(_SLICES_PER_STEP)
    ]
    big = jnp.concatenate(tmps, axis=0)
    out = jnp.dot(big, w_ref[...], preferred_element_type=jnp.float32)
    n = x_ref.shape[1]
    for s in range(_SLICES_PER_STEP):
        o_ref[s] = out[s * n:(s + 1) * n] + b_ref[...]


def kernel(inputs, normalized_adj, weights_0, bias_0):
    b, t, n, d = inputs.shape
    hid = weights_0.shape[1]
    s = _SLICES_PER_STEP
    x = inputs.reshape(b * t, n, d)
    a_bf = normalized_adj.astype(jnp.bfloat16)
    w_bf = weights_0.astype(jnp.bfloat16)
    bias2 = bias_0.reshape(1, hid)

    out = pl.pallas_call(
        _gcn_body,
        grid=(b * t // s,),
        in_specs=[
            pl.BlockSpec((s, n, d), lambda i: (i, 0, 0)),
            pl.BlockSpec((n, n), lambda i: (0, 0)),
            pl.BlockSpec((d, hid), lambda i: (0, 0)),
            pl.BlockSpec((1, hid), lambda i: (0, 0)),
        ],
        out_specs=pl.BlockSpec((s, n, hid), lambda i: (i, 0, 0)),
        out_shape=jax.ShapeDtypeStruct((b * t, n, hid), jnp.float32),
    )(x, a_bf, w_bf, bias2)
    return out.reshape(b, t, n, hid)
